# baseline (speedup 1.0000x reference)
import jax
import jax.numpy as jnp
from jax import lax
from jax.experimental import pallas as pl
from jax.experimental.pallas import tpu as pltpu
from jax.experimental.pallas import tpu_sc as plsc

NC = 2
NS = 16
L = 16
NW = NC * NS

D = 64
K = 10
PAIR = 128
CHUNK = 64
GSLICE = 128
RB = 256
ROWG = 16


def _log1p_small(u):
    z = u / (2.0 + u)
    z2 = z * z
    return 2.0 * z * (1.0 + z2 * (1.0 / 3.0 + z2 * (0.2 + z2 * (1.0 / 7.0))))


def _softplus(t):
    return jnp.maximum(t, 0.0) + _log1p_small(jnp.exp(-jnp.abs(t)))


def _transpose_block(in_v, out_v):
    dvec = lax.iota(jnp.int32, L)
    rvecs = [c * L + dvec for c in range(D // L)]

    @plsc.parallel_loop(0, RB // 2, step=ROWG)
    def grp(p0):
        for pp in range(ROWG):
            col = 2 * (p0 + pp)
            for h in range(2):
                cvec = jnp.full((L,), col + h, jnp.int32)
                for c in range(D // L):
                    vals = plsc.load_gather(in_v, [rvecs[c], cvec])
                    out_v[p0 + pp, pl.ds(h * D + c * L, L)] = vals


def _repack_body(wct_hbm, wxt_hbm, ctail_hbm, xtail_hbm,
                 wcp_hbm, wxp_hbm, in_v, out_v,
                 sem_in0, sem_in1, sem_out0, sem_out1):
    wid = lax.axis_index("s") * NC + lax.axis_index("c")
    vocab = wct_hbm.shape[1]
    nfull = vocab // RB
    tail = vocab - nfull * RB
    nj = (nfull + NW - 1) // NW
    njp = (nj + 1) // 2
    sem_in = (sem_in0, sem_in1)
    sem_out = (sem_out0, sem_out1)

    for src, tailref, dst in ((wct_hbm, ctail_hbm, wcp_hbm),
                              (wxt_hbm, xtail_hbm, wxp_hbm)):
        def start_in(j, p):
            b = wid + j * NW

            @pl.when(b < nfull)
            def _():
                col = pl.multiple_of(b * RB, RB)
                pltpu.async_copy(src.at[:, pl.ds(col, RB)],
                                 in_v.at[p], sem_in[p])

        def wait_in(p):
            pltpu.make_async_copy(src.at[:, pl.ds(0, RB)],
                                  in_v.at[p], sem_in[p]).wait()

        def start_out(j, p):
            b = wid + j * NW
            orow = pl.multiple_of(b * (RB // 2), RB // 2)
            pltpu.async_copy(out_v.at[p],
                             dst.at[pl.ds(orow, RB // 2)], sem_out[p])

        def wait_out(p):
            pltpu.make_async_copy(out_v.at[p],
                                  dst.at[pl.ds(0, RB // 2)],
                                  sem_out[p]).wait()

        start_in(0, 0)
        start_in(1, 1)

        def jpair(j2, carry):
            for p in range(2):
                j = 2 * j2 + p
                b = wid + j * NW

                @pl.when(b < nfull)
                def _():
                    wait_in(p)

                    @pl.when(j2 >= 1)
                    def _():
                        wait_out(p)

                    _transpose_block(in_v.at[p], out_v.at[p])
                    start_out(j, p)
                    start_in(j + 2, p)
            return carry

        lax.fori_loop(0, njp, jpair, 0)
        wait_out(0)
        wait_out(1)

        if tail:
            @pl.when(wid == NW - 1)
            def _():
                pltpu.sync_copy(
                    tailref, dst.at[pl.ds(nfull * (RB // 2), tail // 2)])


def _gather_body(wc_hbm, wx_hbm, cidx_hbm, xidx_hbm, nidx_hbm, out_hbm,
                 cidx_v, xidx_v, nidx_v, hcidx_v, hxidx_v, hnidx_v,
                 crow_v, xrow_v, nrow_v, out_v, sem):
    wid = lax.axis_index("s") * NC + lax.axis_index("c")
    n_per_w = cidx_hbm.shape[0] // NW
    nchunks = n_per_w // CHUNK
    nslices = (CHUNK * K) // GSLICE

    def chunk_body(t, carry):
        base = wid * n_per_w + t * CHUNK
        pltpu.sync_copy(cidx_hbm.at[pl.ds(base, CHUNK)], cidx_v)
        pltpu.sync_copy(xidx_hbm.at[pl.ds(base, CHUNK)], xidx_v)
        pltpu.sync_copy(nidx_hbm.at[pl.ds(base * K, CHUNK * K)], nidx_v)
        for i in range(CHUNK // L):
            s = pl.ds(i * L, L)
            hcidx_v[s] = lax.shift_right_logical(cidx_v[s], 1)
            hxidx_v[s] = lax.shift_right_logical(xidx_v[s], 1)
        for i in range((CHUNK * K) // L):
            s = pl.ds(i * L, L)
            hnidx_v[s] = lax.shift_right_logical(nidx_v[s], 1)
        copies = [
            pltpu.async_copy(wc_hbm.at[hcidx_v], crow_v, sem),
            pltpu.async_copy(wx_hbm.at[hxidx_v], xrow_v, sem),
        ]
        for j in range(nslices):
            copies.append(pltpu.async_copy(
                wx_hbm.at[hnidx_v.at[pl.ds(j * GSLICE, GSLICE)]],
                nrow_v.at[pl.ds(j * GSLICE, GSLICE)], sem))
        for c in copies:
            c.wait()

        for g in range(CHUNK // L):
            gs = pl.ds(g * L, L)
            rows = g * L + lax.iota(jnp.int32, L)
            chalf = lax.shift_left(jnp.bitwise_and(cidx_v[gs], 1), 6)
            xhalf = lax.shift_left(jnp.bitwise_and(xidx_v[gs], 1), 6)
            nrows = [rows * K + k for k in range(K)]
            nhalf = []
            for k in range(K):
                nk = plsc.load_gather(nidx_v, [nrows[k]])
                nhalf.append(lax.shift_left(jnp.bitwise_and(nk, 1), 6))

            def dbody(d, accs):
                cc = plsc.load_gather(crow_v, [rows, chalf + d])
                cx = plsc.load_gather(xrow_v, [rows, xhalf + d])
                new = [accs[0] + cc * cx]
                for k in range(K):
                    cn = plsc.load_gather(nrow_v, [nrows[k], nhalf[k] + d])
                    new.append(accs[k + 1] + cc * cn)
                return tuple(new)

            accs = lax.fori_loop(
                0, D, dbody,
                tuple(jnp.zeros((L,), jnp.float32) for _ in range(K + 1)))
            p = accs[0]
            loss = _softplus(-p) + 0.1 * p
            for k in range(K):
                nk = accs[k + 1]
                loss = loss + _softplus(-nk) + 0.9 * nk
            out_v[gs] = loss

        pltpu.sync_copy(out_v, out_hbm.at[pl.ds(base, CHUNK)])
        return carry

    lax.fori_loop(0, nchunks, chunk_body, 0)


def _make_repack(vocab):
    mesh = plsc.VectorSubcoreMesh(
        core_axis_name="c", subcore_axis_name="s",
        num_cores=NC, num_subcores=NS)
    return pl.kernel(
        _repack_body,
        out_type=(jax.ShapeDtypeStruct((vocab // 2, PAIR), jnp.float32),
                  jax.ShapeDtypeStruct((vocab // 2, PAIR), jnp.float32)),
        mesh=mesh,
        compiler_params=pltpu.CompilerParams(
            needs_layout_passes=False, use_tc_tiling_on_sc=True),
        scratch_types=[
            pltpu.VMEM((2, D, RB), jnp.float32),
            pltpu.VMEM((2, RB // 2, PAIR), jnp.float32),
            pltpu.SemaphoreType.DMA,
            pltpu.SemaphoreType.DMA,
            pltpu.SemaphoreType.DMA,
            pltpu.SemaphoreType.DMA,
        ],
    )


def _make_gather(batch, vocab):
    mesh = plsc.VectorSubcoreMesh(
        core_axis_name="c", subcore_axis_name="s",
        num_cores=NC, num_subcores=NS)
    return pl.kernel(
        _gather_body,
        out_type=jax.ShapeDtypeStruct((batch,), jnp.float32),
        mesh=mesh,
        compiler_params=pltpu.CompilerParams(
            needs_layout_passes=False, use_tc_tiling_on_sc=True),
        scratch_types=[
            pltpu.VMEM((CHUNK,), jnp.int32),
            pltpu.VMEM((CHUNK,), jnp.int32),
            pltpu.VMEM((CHUNK * K,), jnp.int32),
            pltpu.VMEM((CHUNK,), jnp.int32),
            pltpu.VMEM((CHUNK,), jnp.int32),
            pltpu.VMEM((CHUNK * K,), jnp.int32),
            pltpu.VMEM((CHUNK, PAIR), jnp.float32),
            pltpu.VMEM((CHUNK, PAIR), jnp.float32),
            pltpu.VMEM((CHUNK * K, PAIR), jnp.float32),
            pltpu.VMEM((CHUNK,), jnp.float32),
            pltpu.SemaphoreType.DMA,
        ],
    )


def kernel(center, context, negatives, W_center, W_context):
    batch = center.shape[0]
    vocab = W_center.shape[0]
    cidx = center.astype(jnp.int32)
    xidx = context.astype(jnp.int32)
    nidx = negatives.astype(jnp.int32).reshape(-1)
    tail = vocab % RB
    ctail = W_center[vocab - tail:].reshape(tail // 2, PAIR)
    xtail = W_context[vocab - tail:].reshape(tail // 2, PAIR)
    wcp, wxp = _make_repack(vocab)(W_center.T, W_context.T, ctail, xtail)
    return _make_gather(batch, vocab)(wcp, wxp, cidx, xidx, nidx)


# speedup vs baseline: 2.9828x; 2.9828x over previous
import jax
import jax.numpy as jnp
from jax import lax
from jax.experimental import pallas as pl
from jax.experimental.pallas import tpu as pltpu
from jax.experimental.pallas import tpu_sc as plsc

NC = 2
NS = 16
L = 16
NW = NC * NS

D = 64
K = 10
PAIR = 128
CHUNK = 64
GSLICE = 128
RB = 256
ROWG = 16


def _log1p_small(u):
    z = u / (2.0 + u)
    z2 = z * z
    return 2.0 * z * (1.0 + z2 * (1.0 / 3.0 + z2 * (0.2 + z2 * (1.0 / 7.0))))


def _softplus(t):
    return jnp.maximum(t, 0.0) + _log1p_small(jnp.exp(-jnp.abs(t)))


def _transpose_block(in_v, out_v):
    lane = lax.iota(jnp.int32, L)
    dd = jnp.bitwise_and(lane, 3)
    jj = lax.shift_right_logical(lane, 2)
    orow_pat = lax.shift_right_logical(jj, 1)
    ocol_pat = lax.shift_left(jnp.bitwise_and(jj, 1), 6) + dd

    @plsc.parallel_loop(0, RB, step=8)
    def grp(j0):
        p0 = lax.shift_right_logical(j0, 1)
        for js in range(2):
            cvec = jj + (j0 + 4 * js)
            orvec = orow_pat + (p0 + 2 * js)
            for d0 in range(0, D, 4):
                vals = plsc.load_gather(in_v, [dd + d0, cvec])
                plsc.store_scatter(out_v, [orvec, ocol_pat + d0], vals)


def _repack_body(wct_hbm, wxt_hbm, ctail_hbm, xtail_hbm,
                 wcp_hbm, wxp_hbm, in_v, out_v,
                 sem_in0, sem_in1, sem_out0, sem_out1):
    wid = lax.axis_index("s") * NC + lax.axis_index("c")
    vocab = wct_hbm.shape[1]
    nfull = vocab // RB
    tail = vocab - nfull * RB
    nj = (nfull + NW - 1) // NW
    njp = (nj + 1) // 2
    sem_in = (sem_in0, sem_in1)
    sem_out = (sem_out0, sem_out1)

    for src, tailref, dst in ((wct_hbm, ctail_hbm, wcp_hbm),
                              (wxt_hbm, xtail_hbm, wxp_hbm)):
        def start_in(j, p):
            b = wid + j * NW

            @pl.when(b < nfull)
            def _():
                col = pl.multiple_of(b * RB, RB)
                pltpu.async_copy(src.at[:, pl.ds(col, RB)],
                                 in_v.at[p], sem_in[p])

        def wait_in(p):
            pltpu.make_async_copy(src.at[:, pl.ds(0, RB)],
                                  in_v.at[p], sem_in[p]).wait()

        def start_out(j, p):
            b = wid + j * NW
            orow = pl.multiple_of(b * (RB // 2), RB // 2)
            pltpu.async_copy(out_v.at[p],
                             dst.at[pl.ds(orow, RB // 2)], sem_out[p])

        def wait_out(p):
            pltpu.make_async_copy(out_v.at[p],
                                  dst.at[pl.ds(0, RB // 2)],
                                  sem_out[p]).wait()

        start_in(0, 0)
        start_in(1, 1)

        def jpair(j2, carry):
            for p in range(2):
                j = 2 * j2 + p
                b = wid + j * NW

                @pl.when(b < nfull)
                def _():
                    wait_in(p)

                    @pl.when(j2 >= 1)
                    def _():
                        wait_out(p)

                    _transpose_block(in_v.at[p], out_v.at[p])
                    start_out(j, p)
                    start_in(j + 2, p)
            return carry

        lax.fori_loop(0, njp, jpair, 0)
        wait_out(0)
        wait_out(1)

        if tail:
            @pl.when(wid == NW - 1)
            def _():
                pltpu.sync_copy(
                    tailref, dst.at[pl.ds(nfull * (RB // 2), tail // 2)])


def _gather_body(wc_hbm, wx_hbm, cidx_hbm, xidx_hbm, nidx_hbm, out_hbm,
                 cidx_v, xidx_v, nidx_v, hcidx_v, hxidx_v, hnidx_v,
                 crow_v, xrow_v, nrow_v, out_v, sem):
    wid = lax.axis_index("s") * NC + lax.axis_index("c")
    n_per_w = cidx_hbm.shape[0] // NW
    nchunks = n_per_w // CHUNK
    nslices = (CHUNK * K) // GSLICE

    def chunk_body(t, carry):
        base = wid * n_per_w + t * CHUNK
        pltpu.sync_copy(cidx_hbm.at[pl.ds(base, CHUNK)], cidx_v)
        pltpu.sync_copy(xidx_hbm.at[pl.ds(base, CHUNK)], xidx_v)
        pltpu.sync_copy(nidx_hbm.at[pl.ds(base * K, CHUNK * K)], nidx_v)
        for i in range(CHUNK // L):
            s = pl.ds(i * L, L)
            hcidx_v[s] = lax.shift_right_logical(cidx_v[s], 1)
            hxidx_v[s] = lax.shift_right_logical(xidx_v[s], 1)
        for i in range((CHUNK * K) // L):
            s = pl.ds(i * L, L)
            hnidx_v[s] = lax.shift_right_logical(nidx_v[s], 1)
        copies = [
            pltpu.async_copy(wc_hbm.at[hcidx_v], crow_v, sem),
            pltpu.async_copy(wx_hbm.at[hxidx_v], xrow_v, sem),
        ]
        for j in range(nslices):
            copies.append(pltpu.async_copy(
                wx_hbm.at[hnidx_v.at[pl.ds(j * GSLICE, GSLICE)]],
                nrow_v.at[pl.ds(j * GSLICE, GSLICE)], sem))
        for c in copies:
            c.wait()

        for g in range(CHUNK // L):
            gs = pl.ds(g * L, L)
            rows = g * L + lax.iota(jnp.int32, L)
            chalf = lax.shift_left(jnp.bitwise_and(cidx_v[gs], 1), 6)
            xhalf = lax.shift_left(jnp.bitwise_and(xidx_v[gs], 1), 6)
            nrows = [rows * K + k for k in range(K)]
            nhalf = []
            for k in range(K):
                nk = plsc.load_gather(nidx_v, [nrows[k]])
                nhalf.append(lax.shift_left(jnp.bitwise_and(nk, 1), 6))

            def dbody(d, accs):
                cc = plsc.load_gather(crow_v, [rows, chalf + d])
                cx = plsc.load_gather(xrow_v, [rows, xhalf + d])
                new = [accs[0] + cc * cx]
                for k in range(K):
                    cn = plsc.load_gather(nrow_v, [nrows[k], nhalf[k] + d])
                    new.append(accs[k + 1] + cc * cn)
                return tuple(new)

            accs = lax.fori_loop(
                0, D, dbody,
                tuple(jnp.zeros((L,), jnp.float32) for _ in range(K + 1)))
            p = accs[0]
            loss = _softplus(-p) + 0.1 * p
            for k in range(K):
                nk = accs[k + 1]
                loss = loss + _softplus(-nk) + 0.9 * nk
            out_v[gs] = loss

        pltpu.sync_copy(out_v, out_hbm.at[pl.ds(base, CHUNK)])
        return carry

    lax.fori_loop(0, nchunks, chunk_body, 0)


def _make_repack(vocab):
    mesh = plsc.VectorSubcoreMesh(
        core_axis_name="c", subcore_axis_name="s",
        num_cores=NC, num_subcores=NS)
    return pl.kernel(
        _repack_body,
        out_type=(jax.ShapeDtypeStruct((vocab // 2, PAIR), jnp.float32),
                  jax.ShapeDtypeStruct((vocab // 2, PAIR), jnp.float32)),
        mesh=mesh,
        compiler_params=pltpu.CompilerParams(
            needs_layout_passes=False, use_tc_tiling_on_sc=True),
        scratch_types=[
            pltpu.VMEM((2, D, RB), jnp.float32),
            pltpu.VMEM((2, RB // 2, PAIR), jnp.float32),
            pltpu.SemaphoreType.DMA,
            pltpu.SemaphoreType.DMA,
            pltpu.SemaphoreType.DMA,
            pltpu.SemaphoreType.DMA,
        ],
    )


def _make_gather(batch, vocab):
    mesh = plsc.VectorSubcoreMesh(
        core_axis_name="c", subcore_axis_name="s",
        num_cores=NC, num_subcores=NS)
    return pl.kernel(
        _gather_body,
        out_type=jax.ShapeDtypeStruct((batch,), jnp.float32),
        mesh=mesh,
        compiler_params=pltpu.CompilerParams(
            needs_layout_passes=False, use_tc_tiling_on_sc=True),
        scratch_types=[
            pltpu.VMEM((CHUNK,), jnp.int32),
            pltpu.VMEM((CHUNK,), jnp.int32),
            pltpu.VMEM((CHUNK * K,), jnp.int32),
            pltpu.VMEM((CHUNK,), jnp.int32),
            pltpu.VMEM((CHUNK,), jnp.int32),
            pltpu.VMEM((CHUNK * K,), jnp.int32),
            pltpu.VMEM((CHUNK, PAIR), jnp.float32),
            pltpu.VMEM((CHUNK, PAIR), jnp.float32),
            pltpu.VMEM((CHUNK * K, PAIR), jnp.float32),
            pltpu.VMEM((CHUNK,), jnp.float32),
            pltpu.SemaphoreType.DMA,
        ],
    )


def kernel(center, context, negatives, W_center, W_context):
    batch = center.shape[0]
    vocab = W_center.shape[0]
    cidx = center.astype(jnp.int32)
    xidx = context.astype(jnp.int32)
    nidx = negatives.astype(jnp.int32).reshape(-1)
    tail = vocab % RB
    ctail = W_center[vocab - tail:].reshape(tail // 2, PAIR)
    xtail = W_context[vocab - tail:].reshape(tail // 2, PAIR)
    wcp, wxp = _make_repack(vocab)(W_center.T, W_context.T, ctail, xtail)
    return _make_gather(batch, vocab)(wcp, wxp, cidx, xidx, nidx)


# speedup vs baseline: 3.4742x; 1.1647x over previous
import jax
import jax.numpy as jnp
from jax import lax
from jax.experimental import pallas as pl
from jax.experimental.pallas import tpu as pltpu
from jax.experimental.pallas import tpu_sc as plsc

NC = 2
NS = 16
L = 16
NW = NC * NS

D = 64
K = 10
PAIR = 128
CHUNK = 64
GSLICE = 128
RB = 256
ROWG = 16


def _log1p_small(u):
    z = u / (2.0 + u)
    z2 = z * z
    return 2.0 * z * (1.0 + z2 * (1.0 / 3.0 + z2 * (0.2 + z2 * (1.0 / 7.0))))


def _softplus(t):
    return jnp.maximum(t, 0.0) + _log1p_small(jnp.exp(-jnp.abs(t)))


def _transpose_block(in_v, out_v):
    lane = lax.iota(jnp.int32, L)
    dd = jnp.bitwise_and(lane, 3)
    jj = lax.shift_right_logical(lane, 2)
    orow_pat = lax.shift_right_logical(jj, 1)
    ocol_pat = lax.shift_left(jnp.bitwise_and(jj, 1), 6) + dd

    @plsc.parallel_loop(0, RB, step=8)
    def grp(j0):
        p0 = lax.shift_right_logical(j0, 1)
        for js in range(2):
            cvec = jj + (j0 + 4 * js)
            orvec = orow_pat + (p0 + 2 * js)
            for d0 in range(0, D, 4):
                vals = plsc.load_gather(in_v, [dd + d0, cvec])
                plsc.store_scatter(out_v, [orvec, ocol_pat + d0], vals)


def _repack_body(wct_hbm, wxt_hbm, ctail_hbm, xtail_hbm,
                 wcp_hbm, wxp_hbm, in_v, out_v,
                 sem_in0, sem_in1, sem_out0, sem_out1):
    wid = lax.axis_index("s") * NC + lax.axis_index("c")
    vocab = wct_hbm.shape[1]
    nfull = vocab // RB
    tail = vocab - nfull * RB
    nj = (nfull + NW - 1) // NW
    njp = (nj + 1) // 2
    sem_in = (sem_in0, sem_in1)
    sem_out = (sem_out0, sem_out1)

    for src, tailref, dst in ((wct_hbm, ctail_hbm, wcp_hbm),
                              (wxt_hbm, xtail_hbm, wxp_hbm)):
        def start_in(j, p):
            b = wid + j * NW

            @pl.when(b < nfull)
            def _():
                col = pl.multiple_of(b * RB, RB)
                pltpu.async_copy(src.at[:, pl.ds(col, RB)],
                                 in_v.at[p], sem_in[p])

        def wait_in(p):
            pltpu.make_async_copy(src.at[:, pl.ds(0, RB)],
                                  in_v.at[p], sem_in[p]).wait()

        def start_out(j, p):
            b = wid + j * NW
            orow = pl.multiple_of(b * (RB // 2), RB // 2)
            pltpu.async_copy(out_v.at[p],
                             dst.at[pl.ds(orow, RB // 2)], sem_out[p])

        def wait_out(p):
            pltpu.make_async_copy(out_v.at[p],
                                  dst.at[pl.ds(0, RB // 2)],
                                  sem_out[p]).wait()

        start_in(0, 0)
        start_in(1, 1)

        def jpair(j2, carry):
            for p in range(2):
                j = 2 * j2 + p
                b = wid + j * NW

                @pl.when(b < nfull)
                def _():
                    wait_in(p)

                    @pl.when(j2 >= 1)
                    def _():
                        wait_out(p)

                    _transpose_block(in_v.at[p], out_v.at[p])
                    start_out(j, p)
                    start_in(j + 2, p)
            return carry

        lax.fori_loop(0, njp, jpair, 0)
        wait_out(0)
        wait_out(1)

        if tail:
            @pl.when(wid == NW - 1)
            def _():
                pltpu.sync_copy(
                    tailref, dst.at[pl.ds(nfull * (RB // 2), tail // 2)])


def _gather_body(wc_hbm, wx_hbm, cidx_hbm, xidx_hbm, nidx_hbm, out_hbm,
                 cidx_v, xidx_v, nidx_v, hcidx_v, hxidx_v, hnidx_v,
                 crow_v, xrow_v, nrow_v, score_v, out_v, sem):
    wid = lax.axis_index("s") * NC + lax.axis_index("c")
    n_per_w = cidx_hbm.shape[0] // NW
    nchunks = n_per_w // CHUNK
    nslices = (CHUNK * K) // GSLICE

    def chunk_body(t, carry):
        base = wid * n_per_w + t * CHUNK
        pltpu.sync_copy(cidx_hbm.at[pl.ds(base, CHUNK)], cidx_v)
        pltpu.sync_copy(xidx_hbm.at[pl.ds(base, CHUNK)], xidx_v)
        pltpu.sync_copy(nidx_hbm.at[pl.ds(base * K, CHUNK * K)], nidx_v)
        for i in range(CHUNK // L):
            s = pl.ds(i * L, L)
            hcidx_v[s] = lax.shift_right_logical(cidx_v[s], 1)
            hxidx_v[s] = lax.shift_right_logical(xidx_v[s], 1)
        for i in range((CHUNK * K) // L):
            s = pl.ds(i * L, L)
            hnidx_v[s] = lax.shift_right_logical(nidx_v[s], 1)
        copies = [
            pltpu.async_copy(wc_hbm.at[hcidx_v], crow_v, sem),
            pltpu.async_copy(wx_hbm.at[hxidx_v], xrow_v, sem),
        ]
        for j in range(nslices):
            copies.append(pltpu.async_copy(
                wx_hbm.at[hnidx_v.at[pl.ds(j * GSLICE, GSLICE)]],
                nrow_v.at[pl.ds(j * GSLICE, GSLICE)], sem))
        for c in copies:
            c.wait()

        lane = lax.iota(jnp.int32, L)
        jj = lax.shift_right_logical(lane, 2)
        dd = jnp.bitwise_and(lane, 3)

        @plsc.parallel_loop(0, CHUNK // 4)
        def quad(q):
            e = 4 * q + jj
            cv = plsc.load_gather(cidx_v, [e])
            xv = plsc.load_gather(xidx_v, [e])
            ch = lax.shift_left(jnp.bitwise_and(cv, 1), 6)
            xh = lax.shift_left(jnp.bitwise_and(xv, 1), 6)
            nrows, nh = [], []
            for k in range(K):
                nv = plsc.load_gather(nidx_v, [e * K + k])
                nh.append(lax.shift_left(jnp.bitwise_and(nv, 1), 6))
                nrows.append(e * K + k)
            zero = jnp.zeros((L,), jnp.float32)
            accp = zero
            accn = [zero] * K
            for d0 in range(0, D, 4):
                dv = d0 + dd
                cc = plsc.load_gather(crow_v, [e, ch + dv])
                cx = plsc.load_gather(xrow_v, [e, xh + dv])
                accp = accp + cc * cx
                for k in range(K):
                    cn = plsc.load_gather(nrow_v, [nrows[k], nh[k] + dv])
                    accn[k] = accn[k] + cc * cn
            score_v[0, pl.ds(q * L, L)] = accp
            for k in range(K):
                score_v[1 + k, pl.ds(q * L, L)] = accn[k]

        for g in range(CHUNK // L):
            cbase = g * (4 * L) + 4 * lane

            def rsum(a):
                av = jnp.full((L,), a, jnp.int32)
                tot = plsc.load_gather(score_v, [av, cbase])
                for r in range(1, 4):
                    tot = tot + plsc.load_gather(score_v, [av, cbase + r])
                return tot

            p = rsum(0)
            loss = _softplus(-p) + 0.1 * p
            for k in range(K):
                nk = rsum(1 + k)
                loss = loss + _softplus(-nk) + 0.9 * nk
            out_v[pl.ds(g * L, L)] = loss

        pltpu.sync_copy(out_v, out_hbm.at[pl.ds(base, CHUNK)])
        return carry

    lax.fori_loop(0, nchunks, chunk_body, 0)


def _make_repack(vocab):
    mesh = plsc.VectorSubcoreMesh(
        core_axis_name="c", subcore_axis_name="s",
        num_cores=NC, num_subcores=NS)
    return pl.kernel(
        _repack_body,
        out_type=(jax.ShapeDtypeStruct((vocab // 2, PAIR), jnp.float32),
                  jax.ShapeDtypeStruct((vocab // 2, PAIR), jnp.float32)),
        mesh=mesh,
        compiler_params=pltpu.CompilerParams(
            needs_layout_passes=False, use_tc_tiling_on_sc=True),
        scratch_types=[
            pltpu.VMEM((2, D, RB), jnp.float32),
            pltpu.VMEM((2, RB // 2, PAIR), jnp.float32),
            pltpu.SemaphoreType.DMA,
            pltpu.SemaphoreType.DMA,
            pltpu.SemaphoreType.DMA,
            pltpu.SemaphoreType.DMA,
        ],
    )


def _make_gather(batch, vocab):
    mesh = plsc.VectorSubcoreMesh(
        core_axis_name="c", subcore_axis_name="s",
        num_cores=NC, num_subcores=NS)
    return pl.kernel(
        _gather_body,
        out_type=jax.ShapeDtypeStruct((batch,), jnp.float32),
        mesh=mesh,
        compiler_params=pltpu.CompilerParams(
            needs_layout_passes=False, use_tc_tiling_on_sc=True),
        scratch_types=[
            pltpu.VMEM((CHUNK,), jnp.int32),
            pltpu.VMEM((CHUNK,), jnp.int32),
            pltpu.VMEM((CHUNK * K,), jnp.int32),
            pltpu.VMEM((CHUNK,), jnp.int32),
            pltpu.VMEM((CHUNK,), jnp.int32),
            pltpu.VMEM((CHUNK * K,), jnp.int32),
            pltpu.VMEM((CHUNK, PAIR), jnp.float32),
            pltpu.VMEM((CHUNK, PAIR), jnp.float32),
            pltpu.VMEM((CHUNK * K, PAIR), jnp.float32),
            pltpu.VMEM((K + 1, CHUNK * 4), jnp.float32),
            pltpu.VMEM((CHUNK,), jnp.float32),
            pltpu.SemaphoreType.DMA,
        ],
    )


def kernel(center, context, negatives, W_center, W_context):
    batch = center.shape[0]
    vocab = W_center.shape[0]
    cidx = center.astype(jnp.int32)
    xidx = context.astype(jnp.int32)
    nidx = negatives.astype(jnp.int32).reshape(-1)
    tail = vocab % RB
    ctail = W_center[vocab - tail:].reshape(tail // 2, PAIR)
    xtail = W_context[vocab - tail:].reshape(tail // 2, PAIR)
    wcp, wxp = _make_repack(vocab)(W_center.T, W_context.T, ctail, xtail)
    return _make_gather(batch, vocab)(wcp, wxp, cidx, xidx, nidx)


# speedup vs baseline: 3.7188x; 1.0704x over previous
import jax
import jax.numpy as jnp
from jax import lax
from jax.experimental import pallas as pl
from jax.experimental.pallas import tpu as pltpu
from jax.experimental.pallas import tpu_sc as plsc

NC = 2
NS = 16
L = 16
NW = NC * NS

D = 64
K = 10
PAIR = 128
CHUNK = 64
GSLICE = 128
RB = 256
ROWG = 16


def _log1p_small(u):
    z = u / (2.0 + u)
    z2 = z * z
    return 2.0 * z * (1.0 + z2 * (1.0 / 3.0 + z2 * (0.2 + z2 * (1.0 / 7.0))))


def _softplus(t):
    return jnp.maximum(t, 0.0) + _log1p_small(jnp.exp(-jnp.abs(t)))


def _transpose_block(in_v, out_v):
    lane = lax.iota(jnp.int32, L)
    dd = jnp.bitwise_and(lane, 3)
    jj = lax.shift_right_logical(lane, 2)
    orow_pat = lax.shift_right_logical(jj, 1)
    ocol_pat = lax.shift_left(jnp.bitwise_and(jj, 1), 6) + dd

    @plsc.parallel_loop(0, RB, step=8, unroll=2)
    def grp(j0):
        p0 = lax.shift_right_logical(j0, 1)
        for js in range(2):
            cvec = jj + (j0 + 4 * js)
            orvec = orow_pat + (p0 + 2 * js)
            for d0 in range(0, D, 4):
                vals = plsc.load_gather(in_v, [dd + d0, cvec])
                plsc.store_scatter(out_v, [orvec, ocol_pat + d0], vals)


def _repack_body(wct_hbm, wxt_hbm, ctail_hbm, xtail_hbm,
                 wcp_hbm, wxp_hbm, in_v, out_v,
                 sem_in0, sem_in1, sem_out0, sem_out1):
    wid = lax.axis_index("s") * NC + lax.axis_index("c")
    vocab = wct_hbm.shape[1]
    nfull = vocab // RB
    tail = vocab - nfull * RB
    nj = (nfull + NW - 1) // NW
    njp = (nj + 1) // 2
    sem_in = (sem_in0, sem_in1)
    sem_out = (sem_out0, sem_out1)

    for src, tailref, dst in ((wct_hbm, ctail_hbm, wcp_hbm),
                              (wxt_hbm, xtail_hbm, wxp_hbm)):
        def start_in(j, p):
            b = wid + j * NW

            @pl.when(b < nfull)
            def _():
                col = pl.multiple_of(b * RB, RB)
                pltpu.async_copy(src.at[:, pl.ds(col, RB)],
                                 in_v.at[p], sem_in[p])

        def wait_in(p):
            pltpu.make_async_copy(src.at[:, pl.ds(0, RB)],
                                  in_v.at[p], sem_in[p]).wait()

        def start_out(j, p):
            b = wid + j * NW
            orow = pl.multiple_of(b * (RB // 2), RB // 2)
            pltpu.async_copy(out_v.at[p],
                             dst.at[pl.ds(orow, RB // 2)], sem_out[p])

        def wait_out(p):
            pltpu.make_async_copy(out_v.at[p],
                                  dst.at[pl.ds(0, RB // 2)],
                                  sem_out[p]).wait()

        start_in(0, 0)
        start_in(1, 1)

        def jpair(j2, carry):
            for p in range(2):
                j = 2 * j2 + p
                b = wid + j * NW

                @pl.when(b < nfull)
                def _():
                    wait_in(p)

                    @pl.when(j2 >= 1)
                    def _():
                        wait_out(p)

                    _transpose_block(in_v.at[p], out_v.at[p])
                    start_out(j, p)
                    start_in(j + 2, p)
            return carry

        lax.fori_loop(0, njp, jpair, 0)
        wait_out(0)
        wait_out(1)

        if tail:
            @pl.when(wid == NW - 1)
            def _():
                pltpu.sync_copy(
                    tailref, dst.at[pl.ds(nfull * (RB // 2), tail // 2)])


def _gather_body(wc_hbm, wx_hbm, cidx_hbm, xidx_hbm, nidx_hbm, out_hbm,
                 cidx_v, xidx_v, nidx_v, hcidx_v, hxidx_v, hnidx_v,
                 crow_v, xrow_v, nrow_v, score_v, out_v, sem):
    wid = lax.axis_index("s") * NC + lax.axis_index("c")
    n_per_w = cidx_hbm.shape[0] // NW
    nchunks = n_per_w // CHUNK
    nslices = (CHUNK * K) // GSLICE

    def chunk_body(t, carry):
        base = wid * n_per_w + t * CHUNK
        pltpu.sync_copy(cidx_hbm.at[pl.ds(base, CHUNK)], cidx_v)
        pltpu.sync_copy(xidx_hbm.at[pl.ds(base, CHUNK)], xidx_v)
        pltpu.sync_copy(nidx_hbm.at[pl.ds(base * K, CHUNK * K)], nidx_v)
        for i in range(CHUNK // L):
            s = pl.ds(i * L, L)
            hcidx_v[s] = lax.shift_right_logical(cidx_v[s], 1)
            hxidx_v[s] = lax.shift_right_logical(xidx_v[s], 1)
        for i in range((CHUNK * K) // L):
            s = pl.ds(i * L, L)
            hnidx_v[s] = lax.shift_right_logical(nidx_v[s], 1)
        copies = [
            pltpu.async_copy(wc_hbm.at[hcidx_v], crow_v, sem),
            pltpu.async_copy(wx_hbm.at[hxidx_v], xrow_v, sem),
        ]
        for j in range(nslices):
            copies.append(pltpu.async_copy(
                wx_hbm.at[hnidx_v.at[pl.ds(j * GSLICE, GSLICE)]],
                nrow_v.at[pl.ds(j * GSLICE, GSLICE)], sem))
        for c in copies:
            c.wait()

        lane = lax.iota(jnp.int32, L)
        jj = lax.shift_right_logical(lane, 2)
        dd = jnp.bitwise_and(lane, 3)

        @plsc.parallel_loop(0, CHUNK // 4, unroll=2)
        def quad(q):
            e = 4 * q + jj
            cv = plsc.load_gather(cidx_v, [e])
            xv = plsc.load_gather(xidx_v, [e])
            ch = lax.shift_left(jnp.bitwise_and(cv, 1), 6)
            xh = lax.shift_left(jnp.bitwise_and(xv, 1), 6)
            nrows, nh = [], []
            for k in range(K):
                nv = plsc.load_gather(nidx_v, [e * K + k])
                nh.append(lax.shift_left(jnp.bitwise_and(nv, 1), 6))
                nrows.append(e * K + k)
            zero = jnp.zeros((L,), jnp.float32)
            accp = zero
            accn = [zero] * K
            for d0 in range(0, D, 4):
                dv = d0 + dd
                cc = plsc.load_gather(crow_v, [e, ch + dv])
                cx = plsc.load_gather(xrow_v, [e, xh + dv])
                accp = accp + cc * cx
                for k in range(K):
                    cn = plsc.load_gather(nrow_v, [nrows[k], nh[k] + dv])
                    accn[k] = accn[k] + cc * cn
            score_v[0, pl.ds(q * L, L)] = accp
            for k in range(K):
                score_v[1 + k, pl.ds(q * L, L)] = accn[k]

        for g in range(CHUNK // L):
            cbase = g * (4 * L) + 4 * lane

            def rsum(a):
                av = jnp.full((L,), a, jnp.int32)
                tot = plsc.load_gather(score_v, [av, cbase])
                for r in range(1, 4):
                    tot = tot + plsc.load_gather(score_v, [av, cbase + r])
                return tot

            p = rsum(0)
            loss = _softplus(-p) + 0.1 * p
            for k in range(K):
                nk = rsum(1 + k)
                loss = loss + _softplus(-nk) + 0.9 * nk
            out_v[pl.ds(g * L, L)] = loss

        pltpu.sync_copy(out_v, out_hbm.at[pl.ds(base, CHUNK)])
        return carry

    lax.fori_loop(0, nchunks, chunk_body, 0)


def _make_repack(vocab):
    mesh = plsc.VectorSubcoreMesh(
        core_axis_name="c", subcore_axis_name="s",
        num_cores=NC, num_subcores=NS)
    return pl.kernel(
        _repack_body,
        out_type=(jax.ShapeDtypeStruct((vocab // 2, PAIR), jnp.float32),
                  jax.ShapeDtypeStruct((vocab // 2, PAIR), jnp.float32)),
        mesh=mesh,
        compiler_params=pltpu.CompilerParams(
            needs_layout_passes=False, use_tc_tiling_on_sc=True),
        scratch_types=[
            pltpu.VMEM((2, D, RB), jnp.float32),
            pltpu.VMEM((2, RB // 2, PAIR), jnp.float32),
            pltpu.SemaphoreType.DMA,
            pltpu.SemaphoreType.DMA,
            pltpu.SemaphoreType.DMA,
            pltpu.SemaphoreType.DMA,
        ],
    )


def _make_gather(batch, vocab):
    mesh = plsc.VectorSubcoreMesh(
        core_axis_name="c", subcore_axis_name="s",
        num_cores=NC, num_subcores=NS)
    return pl.kernel(
        _gather_body,
        out_type=jax.ShapeDtypeStruct((batch,), jnp.float32),
        mesh=mesh,
        compiler_params=pltpu.CompilerParams(
            needs_layout_passes=False, use_tc_tiling_on_sc=True),
        scratch_types=[
            pltpu.VMEM((CHUNK,), jnp.int32),
            pltpu.VMEM((CHUNK,), jnp.int32),
            pltpu.VMEM((CHUNK * K,), jnp.int32),
            pltpu.VMEM((CHUNK,), jnp.int32),
            pltpu.VMEM((CHUNK,), jnp.int32),
            pltpu.VMEM((CHUNK * K,), jnp.int32),
            pltpu.VMEM((CHUNK, PAIR), jnp.float32),
            pltpu.VMEM((CHUNK, PAIR), jnp.float32),
            pltpu.VMEM((CHUNK * K, PAIR), jnp.float32),
            pltpu.VMEM((K + 1, CHUNK * 4), jnp.float32),
            pltpu.VMEM((CHUNK,), jnp.float32),
            pltpu.SemaphoreType.DMA,
        ],
    )


def kernel(center, context, negatives, W_center, W_context):
    batch = center.shape[0]
    vocab = W_center.shape[0]
    cidx = center.astype(jnp.int32)
    xidx = context.astype(jnp.int32)
    nidx = negatives.astype(jnp.int32).reshape(-1)
    tail = vocab % RB
    ctail = W_center[vocab - tail:].reshape(tail // 2, PAIR)
    xtail = W_context[vocab - tail:].reshape(tail // 2, PAIR)
    wcp, wxp = _make_repack(vocab)(W_center.T, W_context.T, ctail, xtail)
    return _make_gather(batch, vocab)(wcp, wxp, cidx, xidx, nidx)
